# baseline (speedup 1.0000x reference)
import functools

import jax
import jax.numpy as jnp
from jax import lax
from jax.experimental import pallas as pl
from jax.experimental.pallas import tpu as pltpu
from jax.experimental.pallas import tpu_sc as plsc

_W = 64
_D = 64
_B = 16384
_R = _W * _W * _W
_HR = _R // 2
_INFO = plsc.get_sparse_core_info()
_NC, _NS, _L = _INFO.num_cores, _INFO.num_subcores, _INFO.num_lanes
_NW = _NC * _NS
_BPW = _B // _NW
_CHUNK = 128
_NCHUNK = _BPW // _CHUNK

_mesh = plsc.VectorSubcoreMesh(core_axis_name="c", subcore_axis_name="s")


@functools.partial(
    pl.kernel,
    mesh=_mesh,
    compiler_params=pltpu.CompilerParams(
        use_tc_tiling_on_sc=True, needs_layout_passes=False
    ),
    out_type=jax.ShapeDtypeStruct((_D, _B), jnp.float32),
    scratch_types=[
        pltpu.VMEM((_BPW,), jnp.float32),
        pltpu.VMEM((_BPW,), jnp.float32),
        pltpu.VMEM((_BPW,), jnp.float32),
        pltpu.VMEM((_NCHUNK, _CHUNK), jnp.int32),
        pltpu.VMEM((_BPW,), jnp.int32),
        pltpu.VMEM((_D * _L,), jnp.int32),
        pltpu.VMEM((_BPW, 2 * _D), jnp.float32),
        pltpu.VMEM((_D, _BPW), jnp.float32),
        pltpu.SemaphoreType.DMA,
    ],
)
def _lookup(table_hbm, flt_hbm, out_hbm, f0_v, f1_v, f2_v, did_v, half_v,
            skew_v, rows_v, outt_v, sem):
    wid = lax.axis_index("s") * _NC + lax.axis_index("c")
    base = wid * _BPW

    pltpu.sync_copy(flt_hbm.at[pl.ds(base, _BPW)], f0_v)
    pltpu.sync_copy(flt_hbm.at[pl.ds(_B + base, _BPW)], f1_v)
    pltpu.sync_copy(flt_hbm.at[pl.ds(2 * _B + base, _BPW)], f2_v)

    lanes = lax.iota(jnp.int32, _L)
    for c in range(_D):
        skew_v[pl.ds(c * _L, _L)] = (lanes + c) & (_D - 1)

    gathers = []
    for j in range(_NCHUNK):
        for t in range(_CHUNK // _L):
            p = j * _CHUNK + t * _L
            f0 = f0_v[pl.ds(p, _L)]
            f1 = f1_v[pl.ds(p, _L)]
            f2 = f2_v[pl.ds(p, _L)]
            r = ((f0 * float(_W)).astype(jnp.int32) * (_W * _W)
                 + (f1 * float(_W)).astype(jnp.int32) * _W
                 + (f2 * float(_W)).astype(jnp.int32))
            did_v[j, pl.ds(t * _L, _L)] = r & (_HR - 1)
            half_v[pl.ds(p, _L)] = lax.shift_right_logical(r, 17) * _D
        gathers.append(
            pltpu.async_copy(table_hbm.at[did_v.at[j]],
                             rows_v.at[pl.ds(j * _CHUNK, _CHUNK), :], sem))

    for j in range(_NCHUNK):
        gathers[j].wait()

        @pl.loop(j * _CHUNK // _L, (j + 1) * _CHUNK // _L)
        def _extract(g):
            p = g * _L
            slot = p + lanes
            half = half_v[pl.ds(p, _L)]
            for c in range(_D):
                colr = skew_v[pl.ds(c * _L, _L)]
                v = plsc.load_gather(rows_v, [slot, half + colr])
                plsc.store_scatter(outt_v, [colr, slot], v)

    pltpu.sync_copy(outt_v, out_hbm.at[:, pl.ds(base, _BPW)])


def _repack_body(a_ref, b_ref, dst_ref):
    dst_ref[:, pl.ds(0, _D)] = a_ref[...]
    dst_ref[:, pl.ds(_D, _D)] = b_ref[...]


_RB = 8192
_NRB = _HR // _RB


@jax.jit
def _repack(t2):
    return pl.pallas_call(
        _repack_body,
        grid=(_NRB,),
        in_specs=[
            pl.BlockSpec((_RB, _D), lambda i: (i, 0)),
            pl.BlockSpec((_RB, _D), lambda i: (i + _NRB, 0)),
        ],
        out_specs=pl.BlockSpec((_RB, 2 * _D), lambda i: (i, 0)),
        out_shape=jax.ShapeDtypeStruct((_HR, 2 * _D), jnp.float32),
    )(t2, t2)


def kernel(indices, table):
    t128 = _repack(table.reshape(_R, _D))
    flt = indices.T.reshape(-1)
    return _lookup(t128, flt).T


# speedup vs baseline: 1.0642x; 1.0642x over previous
import functools

import jax
import jax.numpy as jnp
from jax import lax
from jax.experimental import pallas as pl
from jax.experimental.pallas import tpu as pltpu
from jax.experimental.pallas import tpu_sc as plsc

_W = 64
_D = 64
_B = 16384
_R = _W * _W * _W
_HR = _R // 2
_INFO = plsc.get_sparse_core_info()
_NC, _NS, _L = _INFO.num_cores, _INFO.num_subcores, _INFO.num_lanes
_NW = _NC * _NS
_BPW = _B // _NW
_CHUNK = 128
_NCHUNK = _BPW // _CHUNK

_mesh = plsc.VectorSubcoreMesh(core_axis_name="c", subcore_axis_name="s")


@functools.partial(
    pl.kernel,
    mesh=_mesh,
    compiler_params=pltpu.CompilerParams(
        use_tc_tiling_on_sc=True, needs_layout_passes=False
    ),
    out_type=jax.ShapeDtypeStruct((_B, 2 * _D), jnp.float32),
    scratch_types=[
        pltpu.VMEM((_BPW,), jnp.float32),
        pltpu.VMEM((_BPW,), jnp.float32),
        pltpu.VMEM((_BPW,), jnp.float32),
        pltpu.VMEM((_NCHUNK, _CHUNK), jnp.int32),
        pltpu.VMEM((_BPW, 2 * _D), jnp.float32),
        pltpu.SemaphoreType.DMA,
    ],
)
def _lookup(table_hbm, flt_hbm, out_hbm, f0_v, f1_v, f2_v, did_v, rows_v, sem):
    wid = lax.axis_index("s") * _NC + lax.axis_index("c")
    base = wid * _BPW

    pltpu.sync_copy(flt_hbm.at[pl.ds(base, _BPW)], f0_v)
    pltpu.sync_copy(flt_hbm.at[pl.ds(_B + base, _BPW)], f1_v)
    pltpu.sync_copy(flt_hbm.at[pl.ds(2 * _B + base, _BPW)], f2_v)

    gathers = []
    for j in range(_NCHUNK):
        for t in range(_CHUNK // _L):
            p = j * _CHUNK + t * _L
            f0 = f0_v[pl.ds(p, _L)]
            f1 = f1_v[pl.ds(p, _L)]
            f2 = f2_v[pl.ds(p, _L)]
            r = ((f0 * float(_W)).astype(jnp.int32) * (_W * _W)
                 + (f1 * float(_W)).astype(jnp.int32) * _W
                 + (f2 * float(_W)).astype(jnp.int32))
            did_v[j, pl.ds(t * _L, _L)] = r & (_HR - 1)
        gathers.append(
            pltpu.async_copy(table_hbm.at[did_v.at[j]],
                             rows_v.at[pl.ds(j * _CHUNK, _CHUNK), :], sem))
    for cp in gathers:
        cp.wait()

    pltpu.sync_copy(rows_v, out_hbm.at[pl.ds(base, _BPW), :])


def _repack_body(a_ref, b_ref, dst_ref):
    dst_ref[:, pl.ds(0, _D)] = a_ref[...]
    dst_ref[:, pl.ds(_D, _D)] = b_ref[...]


_RB = 8192
_NRB = _HR // _RB


@jax.jit
def _repack(t2):
    return pl.pallas_call(
        _repack_body,
        grid=(_NRB,),
        in_specs=[
            pl.BlockSpec((_RB, _D), lambda i: (i, 0)),
            pl.BlockSpec((_RB, _D), lambda i: (i + _NRB, 0)),
        ],
        out_specs=pl.BlockSpec((_RB, 2 * _D), lambda i: (i, 0)),
        out_shape=jax.ShapeDtypeStruct((_HR, 2 * _D), jnp.float32),
    )(t2, t2)


def _select_body(c0_ref, pairs_ref, out_ref):
    sel = c0_ref[...] >= 0.5
    x = jnp.where(sel, pairs_ref[:, pl.ds(_D, _D)],
                  pairs_ref[:, pl.ds(0, _D)])
    out_ref[...] = x.T


_SB = 2048
_NSB = _B // _SB


@jax.jit
def _select(c0, pairs):
    return pl.pallas_call(
        _select_body,
        grid=(_NSB,),
        in_specs=[
            pl.BlockSpec((_SB, 1), lambda i: (i, 0)),
            pl.BlockSpec((_SB, 2 * _D), lambda i: (i, 0)),
        ],
        out_specs=pl.BlockSpec((_D, _SB), lambda i: (0, i)),
        out_shape=jax.ShapeDtypeStruct((_D, _B), jnp.float32),
    )(c0, pairs)


def kernel(indices, table):
    t128 = _repack(table.reshape(_R, _D))
    flt = indices.T.reshape(-1)
    pairs = _lookup(t128, flt)
    return _select(flt[:_B].reshape(_B, 1), pairs).T
